# Initial kernel scaffold; baseline (speedup 1.0000x reference)
import jax
import jax.numpy as jnp
from jax.experimental import pallas as pl


def kernel(query, keys, pool, W_out):
    raise NotImplementedError("write your pallas kernel here")


# speedup vs baseline: 90.8548x; 90.8548x over previous
import functools

import jax
import jax.numpy as jnp
from jax import lax
from jax.experimental import pallas as pl
from jax.experimental.pallas import tpu as pltpu
from jax.experimental.pallas import tpu_sc as plsc

Q = 1024
D = 128
PD = 256
K = 32
N = 100000
SEG = 128
NSEG = 784
NPAD = NSEG * SEG
CHUNK = 512
NCHUNK = NPAD // CHUNK
SPC = CHUNK // SEG

NEGF = -1e30
IBIG = 2 ** 30

NW = 32

QB2 = 256
QB4 = 256
QB6 = 128


def _s1_body(q_ref, k_ref, scores_ref, segmax_ref):
    c = pl.program_id(0)
    s = lax.dot_general(q_ref[...], k_ref[...], (((1,), (1,)), ((), ())),
                        preferred_element_type=jnp.float32)
    col = c * CHUNK + lax.broadcasted_iota(jnp.int32, (Q, CHUNK), 1)
    s = jnp.where(col < N, s, NEGF)
    scores_ref[...] = s
    segmax_ref[0] = jnp.max(s.reshape(Q, SPC, SEG), axis=-1)


def _stage1(q2, keys_p):
    return pl.pallas_call(
        _s1_body,
        grid=(NCHUNK,),
        in_specs=[
            pl.BlockSpec((Q, D), lambda c: (0, 0)),
            pl.BlockSpec((CHUNK, D), lambda c: (c, 0)),
        ],
        out_specs=[
            pl.BlockSpec((Q, CHUNK), lambda c: (0, c)),
            pl.BlockSpec((1, Q, SPC), lambda c: (c, 0, 0)),
        ],
        out_shape=[
            jax.ShapeDtypeStruct((Q, NPAD), jnp.float32),
            jax.ShapeDtypeStruct((NCHUNK, Q, SPC), jnp.float32),
        ],
    )(q2, keys_p)


def _s2_body(sm_ref, seg_ref, absrow_ref):
    b = pl.program_id(0)
    v = sm_ref[...]
    segid = lax.broadcasted_iota(jnp.int32, (QB2, NSEG), 1)
    kiota = lax.broadcasted_iota(jnp.int32, (QB2, K), 1)
    seg_acc = jnp.zeros((QB2, K), jnp.int32)
    for k in range(K):
        m = jnp.max(v, axis=1, keepdims=True)
        sel = jnp.min(jnp.where(v == m, segid, IBIG), axis=1,
                      keepdims=True)
        v = jnp.where(segid == sel, NEGF, v)
        seg_acc = jnp.where(kiota == k, sel, seg_acc)
    qglob = b * QB2 + lax.broadcasted_iota(jnp.int32, (QB2, K), 0)
    seg_ref[...] = seg_acc
    absrow_ref[...] = qglob * NSEG + seg_acc


def _stage2(segmax):
    return pl.pallas_call(
        _s2_body,
        grid=(Q // QB2,),
        in_specs=[pl.BlockSpec((QB2, NSEG), lambda b: (b, 0))],
        out_specs=[
            pl.BlockSpec((QB2, K), lambda b: (b, 0)),
            pl.BlockSpec((QB2, K), lambda b: (b, 0)),
        ],
        out_shape=[
            jax.ShapeDtypeStruct((Q, K), jnp.int32),
            jax.ShapeDtypeStruct((Q, K), jnp.int32),
        ],
    )(segmax)


def _sc_gather(table, idx3, rows_per_w, row_w, out_rows):
    nchunks = rows_per_w // 128
    mesh = plsc.VectorSubcoreMesh(core_axis_name="c", subcore_axis_name="s")

    @functools.partial(
        pl.kernel,
        mesh=mesh,
        out_type=jax.ShapeDtypeStruct((out_rows, row_w), jnp.float32),
        scratch_types=[
            pltpu.VMEM((nchunks, 128), jnp.int32),
            pltpu.VMEM((128, row_w), jnp.float32),
            pltpu.SemaphoreType.DMA,
        ],
    )
    def k(table_hbm, idx_hbm, out_hbm, idx_v, rows_v, sem):
        wid = lax.axis_index("s") * 2 + lax.axis_index("c")
        pltpu.sync_copy(idx_hbm.at[wid], idx_v)
        base = wid * rows_per_w
        for c in range(nchunks):
            pltpu.async_copy(table_hbm.at[idx_v.at[c]], rows_v, sem).wait()
            pltpu.sync_copy(rows_v, out_hbm.at[pl.ds(base + c * 128, 128)])

    return k(table, idx3)


def _s4_body(cand_ref, seg_ref, w_ref, idx_ref):
    v = cand_ref[...]
    lane = lax.broadcasted_iota(jnp.int32, (QB4, K * SEG), 1)
    ej = lax.broadcasted_iota(jnp.int32, (K, K * SEG), 1) // SEG
    ei = lax.broadcasted_iota(jnp.int32, (K, K * SEG), 0)
    emat = jnp.where(ej == ei, 1.0, 0.0).astype(jnp.float32)
    segf = lax.dot_general(seg_ref[...].astype(jnp.float32), emat,
                           (((1,), (0,)), ((), ())),
                           preferred_element_type=jnp.float32,
                           precision=lax.Precision.HIGHEST)
    gidx = (segf + 0.5).astype(jnp.int32) * SEG + lane % SEG

    kiota = lax.broadcasted_iota(jnp.int32, (QB4, K), 1)
    val_acc = jnp.zeros((QB4, K), jnp.float32)
    idx_acc = jnp.zeros((QB4, K), jnp.int32)
    for k in range(K):
        m = jnp.max(v, axis=1, keepdims=True)
        sel = jnp.min(jnp.where(v == m, gidx, IBIG), axis=1, keepdims=True)
        v = jnp.where(gidx == sel, NEGF, v)
        val_acc = jnp.where(kiota == k, m, val_acc)
        idx_acc = jnp.where(kiota == k, sel, idx_acc)
    e = jnp.exp(val_acc - jnp.max(val_acc, axis=1, keepdims=True))
    w_ref[...] = e / jnp.sum(e, axis=1, keepdims=True)
    idx_ref[...] = idx_acc


def _stage4(cand, segsel):
    return pl.pallas_call(
        _s4_body,
        grid=(Q // QB4,),
        in_specs=[
            pl.BlockSpec((QB4, K * SEG), lambda b: (b, 0)),
            pl.BlockSpec((QB4, K), lambda b: (b, 0)),
        ],
        out_specs=[
            pl.BlockSpec((QB4, K), lambda b: (b, 0)),
            pl.BlockSpec((QB4, K), lambda b: (b, 0)),
        ],
        out_shape=[
            jax.ShapeDtypeStruct((Q, K), jnp.float32),
            jax.ShapeDtypeStruct((Q, K), jnp.int32),
        ],
    )(cand, segsel)


def _s6_body(w_ref, g_ref, wout_ref, out_ref):
    w = w_ref[...]
    g = g_ref[...]
    agg = jnp.sum(w[:, :, None] * g, axis=1)
    out_ref[...] = lax.dot_general(agg, wout_ref[...],
                                   (((1,), (1,)), ((), ())),
                                   preferred_element_type=jnp.float32)


def _stage6(weights, gathered, w_out):
    return pl.pallas_call(
        _s6_body,
        grid=(Q // QB6,),
        in_specs=[
            pl.BlockSpec((QB6, K), lambda b: (b, 0)),
            pl.BlockSpec((QB6, K, PD), lambda b: (b, 0, 0)),
            pl.BlockSpec((PD, PD), lambda b: (0, 0)),
        ],
        out_specs=pl.BlockSpec((QB6, PD), lambda b: (b, 0)),
        out_shape=jax.ShapeDtypeStruct((Q, PD), jnp.float32),
    )(weights, gathered, w_out)


def kernel(query, keys, pool, W_out):
    B, S, _ = query.shape
    q2 = query.reshape(Q, D)
    keys_p = jnp.pad(keys, ((0, NPAD - N), (0, 0)))

    scores, segmax3 = _stage1(q2, keys_p)
    segmax = jnp.transpose(segmax3, (1, 0, 2)).reshape(Q, NSEG)

    segsel, absrow = _stage2(segmax)

    table = scores.reshape(Q * NSEG, SEG)
    idx3 = absrow.reshape(NW, (Q * K // NW) // 128, 128)
    cand = _sc_gather(table, idx3, Q * K // NW, SEG, Q * K)

    weights, topidx = _stage4(cand.reshape(Q, K * SEG), segsel)

    pidx3 = topidx.reshape(NW, (Q * K // NW) // 128, 128)
    gathered = _sc_gather(pool, pidx3, Q * K // NW, PD, Q * K)

    out = _stage6(weights, gathered.reshape(Q, K, PD), W_out)
    return out.reshape(B, S, PD)


# speedup vs baseline: 105.2099x; 1.1580x over previous
import functools

import jax
import jax.numpy as jnp
from jax import lax
from jax.experimental import pallas as pl
from jax.experimental.pallas import tpu as pltpu
from jax.experimental.pallas import tpu_sc as plsc

Q = 1024
D = 128
PD = 256
K = 32
N = 100000
SEG = 128
NSEG = 784
NPAD = NSEG * SEG
CHUNK = 1024
NCHUNK = NPAD // CHUNK
SPC = CHUNK // SEG

NEGF = -1e30
IBIG = 2 ** 30

NW = 32

QB2 = 256
QB4 = 256
QB6 = 128


def _s1_body(q_ref, k_ref, scores_ref, segmax_ref):
    c = pl.program_id(0)
    s = lax.dot_general(q_ref[...], k_ref[...], (((1,), (1,)), ((), ())),
                        preferred_element_type=jnp.float32)
    col = c * CHUNK + lax.broadcasted_iota(jnp.int32, (Q, CHUNK), 1)
    s = jnp.where(col < N, s, NEGF)
    s3 = s.reshape(Q, SPC, SEG)
    scores_ref[...] = s3
    segmax_ref[0] = jnp.max(s3, axis=-1)


def _stage1(q2, keys_p):
    return pl.pallas_call(
        _s1_body,
        grid=(NCHUNK,),
        in_specs=[
            pl.BlockSpec((Q, D), lambda c: (0, 0)),
            pl.BlockSpec((CHUNK, D), lambda c: (c, 0)),
        ],
        out_specs=[
            pl.BlockSpec((Q, SPC, SEG), lambda c: (0, c, 0)),
            pl.BlockSpec((1, Q, SPC), lambda c: (c, 0, 0)),
        ],
        out_shape=[
            jax.ShapeDtypeStruct((Q, NSEG, SEG), jnp.float32),
            jax.ShapeDtypeStruct((NCHUNK, Q, SPC), jnp.float32),
        ],
    )(q2, keys_p)


def _s2_body(sm_ref, seg_ref, absrow_ref):
    b = pl.program_id(0)
    v = sm_ref[...]
    segid = lax.broadcasted_iota(jnp.int32, (QB2, NSEG), 1)
    kiota = lax.broadcasted_iota(jnp.int32, (QB2, K), 1)
    seg_acc = jnp.zeros((QB2, K), jnp.int32)
    for k in range(K):
        m = jnp.max(v, axis=1, keepdims=True)
        sel = jnp.min(jnp.where(v == m, segid, IBIG), axis=1,
                      keepdims=True)
        v = jnp.where(segid == sel, NEGF, v)
        seg_acc = jnp.where(kiota == k, sel, seg_acc)
    qglob = b * QB2 + lax.broadcasted_iota(jnp.int32, (QB2, K), 0)
    seg_ref[...] = seg_acc
    absrow_ref[...] = qglob * NSEG + seg_acc


def _stage2(segmax):
    return pl.pallas_call(
        _s2_body,
        grid=(Q // QB2,),
        in_specs=[pl.BlockSpec((QB2, NSEG), lambda b: (b, 0))],
        out_specs=[
            pl.BlockSpec((QB2, K), lambda b: (b, 0)),
            pl.BlockSpec((QB2, K), lambda b: (b, 0)),
        ],
        out_shape=[
            jax.ShapeDtypeStruct((Q, K), jnp.int32),
            jax.ShapeDtypeStruct((Q, K), jnp.int32),
        ],
    )(segmax)


def _sc_gather(table, idx3, rows_per_w, row_w, out_rows):
    nchunks = rows_per_w // 128
    mesh = plsc.VectorSubcoreMesh(core_axis_name="c", subcore_axis_name="s")

    @functools.partial(
        pl.kernel,
        mesh=mesh,
        out_type=jax.ShapeDtypeStruct((out_rows, row_w), jnp.float32),
        scratch_types=[
            pltpu.VMEM((nchunks, 128), jnp.int32),
            pltpu.VMEM((128, row_w), jnp.float32),
            pltpu.SemaphoreType.DMA,
        ],
    )
    def k(table_hbm, idx_hbm, out_hbm, idx_v, rows_v, sem):
        wid = lax.axis_index("s") * 2 + lax.axis_index("c")
        pltpu.sync_copy(idx_hbm.at[wid], idx_v)
        base = wid * rows_per_w
        for c in range(nchunks):
            pltpu.async_copy(table_hbm.at[idx_v.at[c]], rows_v, sem).wait()
            pltpu.sync_copy(rows_v, out_hbm.at[pl.ds(base + c * 128, 128)])

    return k(table, idx3)


def _s4_body(cand_ref, seg_ref, w_ref, idx_ref):
    v = cand_ref[...]
    gidx = (seg_ref[...][:, :, None] * SEG
            + lax.broadcasted_iota(jnp.int32, (QB4, K, SEG), 2))

    kiota = lax.broadcasted_iota(jnp.int32, (QB4, K), 1)
    val_acc = jnp.zeros((QB4, K), jnp.float32)
    idx_acc = jnp.zeros((QB4, K), jnp.int32)
    for k in range(K):
        m = jnp.max(jnp.max(v, axis=2), axis=1, keepdims=True)
        g = jnp.where(v == m[:, :, None], gidx, IBIG)
        sel = jnp.min(jnp.min(g, axis=2), axis=1, keepdims=True)
        v = jnp.where(gidx == sel[:, :, None], NEGF, v)
        val_acc = jnp.where(kiota == k, m, val_acc)
        idx_acc = jnp.where(kiota == k, sel, idx_acc)
    e = jnp.exp(val_acc - jnp.max(val_acc, axis=1, keepdims=True))
    w_ref[...] = e / jnp.sum(e, axis=1, keepdims=True)
    idx_ref[...] = idx_acc


def _stage4(cand, segsel):
    return pl.pallas_call(
        _s4_body,
        grid=(Q // QB4,),
        in_specs=[
            pl.BlockSpec((QB4, K, SEG), lambda b: (b, 0, 0)),
            pl.BlockSpec((QB4, K), lambda b: (b, 0)),
        ],
        out_specs=[
            pl.BlockSpec((QB4, K), lambda b: (b, 0)),
            pl.BlockSpec((QB4, K), lambda b: (b, 0)),
        ],
        out_shape=[
            jax.ShapeDtypeStruct((Q, K), jnp.float32),
            jax.ShapeDtypeStruct((Q, K), jnp.int32),
        ],
    )(cand, segsel)


def _s6_body(w_ref, g_ref, wout_ref, out_ref):
    w = w_ref[...]
    g = g_ref[...]
    agg = jnp.sum(w[:, :, None] * g, axis=1)
    out_ref[...] = lax.dot_general(agg, wout_ref[...],
                                   (((1,), (1,)), ((), ())),
                                   preferred_element_type=jnp.float32)


def _stage6(weights, gathered, w_out):
    return pl.pallas_call(
        _s6_body,
        grid=(Q // QB6,),
        in_specs=[
            pl.BlockSpec((QB6, K), lambda b: (b, 0)),
            pl.BlockSpec((QB6, K, PD), lambda b: (b, 0, 0)),
            pl.BlockSpec((PD, PD), lambda b: (0, 0)),
        ],
        out_specs=pl.BlockSpec((QB6, PD), lambda b: (b, 0)),
        out_shape=jax.ShapeDtypeStruct((Q, PD), jnp.float32),
    )(weights, gathered, w_out)


def kernel(query, keys, pool, W_out):
    B, S, _ = query.shape
    q2 = query.reshape(Q, D)

    scores, segmax3 = _stage1(q2, keys)
    segmax = jnp.transpose(segmax3, (1, 0, 2)).reshape(Q, NSEG)

    segsel, absrow = _stage2(segmax)

    table = scores.reshape(Q * NSEG, SEG)
    idx3 = absrow.reshape(NW, (Q * K // NW) // 128, 128)
    cand = _sc_gather(table, idx3, Q * K // NW, SEG, Q * K)

    weights, topidx = _stage4(cand.reshape(Q, K, SEG), segsel)

    pidx3 = topidx.reshape(NW, (Q * K // NW) // 128, 128)
    gathered = _sc_gather(pool, pidx3, Q * K // NW, PD, Q * K)

    out = _stage6(weights, gathered.reshape(Q, K, PD), W_out)
    return out.reshape(B, S, PD)


# speedup vs baseline: 105.2144x; 1.0000x over previous
import functools

import jax
import jax.numpy as jnp
from jax import lax
from jax.experimental import pallas as pl
from jax.experimental.pallas import tpu as pltpu
from jax.experimental.pallas import tpu_sc as plsc

Q = 1024
D = 128
PD = 256
K = 32
N = 100000
SEG = 128
NSEG = 784
NPAD = NSEG * SEG
CHUNK = 1024
NCHUNK = NPAD // CHUNK
SPC = CHUNK // SEG

NEGF = -1e30
IBIG = 2 ** 30

NW = 32

QB2 = 256
QB4 = 256
QB6 = 128


def _s1_body(q_ref, k_ref, scores_ref, segmax_ref):
    c = pl.program_id(0)
    s = lax.dot_general(q_ref[...], k_ref[...], (((1,), (1,)), ((), ())),
                        preferred_element_type=jnp.float32)
    col = c * CHUNK + lax.broadcasted_iota(jnp.int32, (Q, CHUNK), 1)
    s = jnp.where(col < N, s, NEGF)
    s3 = s.reshape(Q, SPC, SEG)
    scores_ref[...] = s3
    segmax_ref[0] = jnp.max(s3, axis=-1)


def _stage1(q2, keys_p):
    return pl.pallas_call(
        _s1_body,
        grid=(NCHUNK,),
        in_specs=[
            pl.BlockSpec((Q, D), lambda c: (0, 0)),
            pl.BlockSpec((CHUNK, D), lambda c: (c, 0)),
        ],
        out_specs=[
            pl.BlockSpec((Q, SPC, SEG), lambda c: (0, c, 0)),
            pl.BlockSpec((1, Q, SPC), lambda c: (c, 0, 0)),
        ],
        out_shape=[
            jax.ShapeDtypeStruct((Q, NSEG, SEG), jnp.float32),
            jax.ShapeDtypeStruct((NCHUNK, Q, SPC), jnp.float32),
        ],
    )(q2, keys_p)


def _s2_body(sm_ref, seg_ref, absrow_ref):
    b = pl.program_id(0)
    v = sm_ref[...]
    segid = lax.broadcasted_iota(jnp.int32, (QB2, NSEG), 1)
    kiota = lax.broadcasted_iota(jnp.int32, (QB2, K), 1)
    seg_acc = jnp.zeros((QB2, K), jnp.int32)
    for k in range(K):
        m = jnp.max(v, axis=1, keepdims=True)
        sel = jnp.min(jnp.where(v == m, segid, IBIG), axis=1,
                      keepdims=True)
        v = jnp.where(segid == sel, NEGF, v)
        seg_acc = jnp.where(kiota == k, sel, seg_acc)
    qglob = b * QB2 + lax.broadcasted_iota(jnp.int32, (QB2, K), 0)
    seg_ref[...] = seg_acc
    absrow_ref[...] = qglob * NSEG + seg_acc


def _stage2(segmax):
    return pl.pallas_call(
        _s2_body,
        grid=(Q // QB2,),
        in_specs=[pl.BlockSpec((QB2, NSEG), lambda b: (b, 0))],
        out_specs=[
            pl.BlockSpec((QB2, K), lambda b: (b, 0)),
            pl.BlockSpec((QB2, K), lambda b: (b, 0)),
        ],
        out_shape=[
            jax.ShapeDtypeStruct((Q, K), jnp.int32),
            jax.ShapeDtypeStruct((Q, K), jnp.int32),
        ],
    )(segmax)


def _sc_gather(table, idx3, rows_per_w, row_w, out_rows, dtype):
    nchunks = rows_per_w // 128
    mesh = plsc.VectorSubcoreMesh(core_axis_name="c", subcore_axis_name="s")

    @functools.partial(
        pl.kernel,
        mesh=mesh,
        out_type=jax.ShapeDtypeStruct((out_rows, row_w), dtype),
        scratch_types=[
            pltpu.VMEM((nchunks, 128), jnp.int32),
            pltpu.VMEM((128, row_w), dtype),
            pltpu.SemaphoreType.DMA,
        ],
    )
    def k(table_hbm, idx_hbm, out_hbm, idx_v, rows_v, sem):
        wid = lax.axis_index("s") * 2 + lax.axis_index("c")
        pltpu.sync_copy(idx_hbm.at[wid], idx_v)
        base = wid * rows_per_w
        for c in range(nchunks):
            pltpu.async_copy(table_hbm.at[idx_v.at[c]], rows_v, sem).wait()
            pltpu.sync_copy(rows_v, out_hbm.at[pl.ds(base + c * 128, 128)])

    return k(table, idx3)


def _s4_body(cand_ref, seg_ref, w_ref, idx_ref):
    v = cand_ref[...]
    gidx = (seg_ref[...][:, :, None] * SEG
            + lax.broadcasted_iota(jnp.int32, (QB4, K, SEG), 2))

    kiota = lax.broadcasted_iota(jnp.int32, (QB4, K), 1)
    val_acc = jnp.zeros((QB4, K), jnp.float32)
    idx_acc = jnp.zeros((QB4, K), jnp.int32)
    for k in range(K):
        m = jnp.max(jnp.max(v, axis=2), axis=1, keepdims=True)
        g = jnp.where(v == m[:, :, None], gidx, IBIG)
        sel = jnp.min(jnp.min(g, axis=2), axis=1, keepdims=True)
        v = jnp.where(gidx == sel[:, :, None], NEGF, v)
        val_acc = jnp.where(kiota == k, m, val_acc)
        idx_acc = jnp.where(kiota == k, sel, idx_acc)
    e = jnp.exp(val_acc - jnp.max(val_acc, axis=1, keepdims=True))
    w_ref[...] = e / jnp.sum(e, axis=1, keepdims=True)
    idx_ref[...] = idx_acc


def _stage4(cand, segsel):
    return pl.pallas_call(
        _s4_body,
        grid=(Q // QB4,),
        in_specs=[
            pl.BlockSpec((QB4, K, SEG), lambda b: (b, 0, 0)),
            pl.BlockSpec((QB4, K), lambda b: (b, 0)),
        ],
        out_specs=[
            pl.BlockSpec((QB4, K), lambda b: (b, 0)),
            pl.BlockSpec((QB4, K), lambda b: (b, 0)),
        ],
        out_shape=[
            jax.ShapeDtypeStruct((Q, K), jnp.float32),
            jax.ShapeDtypeStruct((Q, K), jnp.int32),
        ],
    )(cand, segsel)


def _s6_body(w_ref, g_ref, wout_ref, out_ref):
    w = w_ref[...]
    g = g_ref[...]
    agg = jnp.sum(w[:, :, None] * g, axis=1)
    out_ref[...] = lax.dot_general(agg, wout_ref[...],
                                   (((1,), (1,)), ((), ())),
                                   preferred_element_type=jnp.float32)


def _stage6(weights, gathered, w_out):
    return pl.pallas_call(
        _s6_body,
        grid=(Q // QB6,),
        in_specs=[
            pl.BlockSpec((QB6, K), lambda b: (b, 0)),
            pl.BlockSpec((QB6, K, PD), lambda b: (b, 0, 0)),
            pl.BlockSpec((PD, PD), lambda b: (0, 0)),
        ],
        out_specs=pl.BlockSpec((QB6, PD), lambda b: (b, 0)),
        out_shape=jax.ShapeDtypeStruct((Q, PD), jnp.float32),
    )(weights, gathered, w_out)


def kernel(query, keys, pool, W_out):
    B, S, _ = query.shape
    q2 = query.reshape(Q, D)

    scores, segmax3 = _stage1(q2, keys)
    segmax = jnp.transpose(segmax3, (1, 0, 2)).reshape(Q, NSEG)

    segsel, absrow = _stage2(segmax)

    table = scores.reshape(Q * NSEG, SEG)
    idx3 = absrow.reshape(NW, (Q * K // NW) // 128, 128)
    cand = _sc_gather(table, idx3, Q * K // NW, SEG, Q * K,
                      jnp.float32)

    weights, topidx = _stage4(cand.reshape(Q, K, SEG), segsel)

    pidx3 = topidx.reshape(NW, (Q * K // NW) // 128, 128)
    gathered = _sc_gather(pool, pidx3, Q * K // NW, PD, Q * K, jnp.float32)

    out = _stage6(weights, gathered.reshape(Q, K, PD), W_out)
    return out.reshape(B, S, PD)


# speedup vs baseline: 141.3594x; 1.3435x over previous
import functools

import jax
import jax.numpy as jnp
from jax import lax
from jax.experimental import pallas as pl
from jax.experimental.pallas import tpu as pltpu
from jax.experimental.pallas import tpu_sc as plsc

Q = 1024
D = 128
PD = 256
K = 32
N = 100000
SEG = 128
NSEG = 784
NPAD = NSEG * SEG
CHUNK = 1024
NCHUNK = NPAD // CHUNK
SPC = CHUNK // SEG

NEGF = -1e30
IBIG = 2 ** 30

NW = 32

QB2 = 256
QB4 = 256
QB6 = 128


def _s1_body(q_ref, k_ref, scores_ref, segmax_ref):
    c = pl.program_id(0)
    s = lax.dot_general(q_ref[...], k_ref[...], (((1,), (1,)), ((), ())),
                        preferred_element_type=jnp.float32)
    col = c * CHUNK + lax.broadcasted_iota(jnp.int32, (Q, CHUNK), 1)
    s = jnp.where(col < N, s, NEGF)
    s3 = s.reshape(Q, SPC, SEG)
    scores_ref[...] = s3
    segmax_ref[0] = jnp.max(s3, axis=-1)


def _stage1(q2, keys_p):
    return pl.pallas_call(
        _s1_body,
        grid=(NCHUNK,),
        in_specs=[
            pl.BlockSpec((Q, D), lambda c: (0, 0)),
            pl.BlockSpec((CHUNK, D), lambda c: (c, 0)),
        ],
        out_specs=[
            pl.BlockSpec((Q, SPC, SEG), lambda c: (0, c, 0)),
            pl.BlockSpec((1, Q, SPC), lambda c: (c, 0, 0)),
        ],
        out_shape=[
            jax.ShapeDtypeStruct((Q, NSEG, SEG), jnp.float32),
            jax.ShapeDtypeStruct((NCHUNK, Q, SPC), jnp.float32),
        ],
    )(q2, keys_p)


def _s2_body(sm_ref, seg_ref, absrow_ref):
    b = pl.program_id(0)
    v = sm_ref[...]
    segid = lax.broadcasted_iota(jnp.int32, (QB2, NSEG), 1)
    kiota = lax.broadcasted_iota(jnp.int32, (QB2, K), 1)
    seg_acc = jnp.zeros((QB2, K), jnp.int32)
    for k in range(K):
        m = jnp.max(v, axis=1, keepdims=True)
        sel = jnp.min(jnp.where(v == m, segid, IBIG), axis=1,
                      keepdims=True)
        v = jnp.where(segid == sel, NEGF, v)
        seg_acc = jnp.where(kiota == k, sel, seg_acc)
    qglob = b * QB2 + lax.broadcasted_iota(jnp.int32, (QB2, K), 0)
    seg_ref[...] = seg_acc
    absrow_ref[...] = qglob * NSEG + seg_acc


def _stage2(segmax):
    return pl.pallas_call(
        _s2_body,
        grid=(Q // QB2,),
        in_specs=[pl.BlockSpec((QB2, NSEG), lambda b: (b, 0))],
        out_specs=[
            pl.BlockSpec((QB2, K), lambda b: (b, 0)),
            pl.BlockSpec((QB2, K), lambda b: (b, 0)),
        ],
        out_shape=[
            jax.ShapeDtypeStruct((Q, K), jnp.int32),
            jax.ShapeDtypeStruct((Q, K), jnp.int32),
        ],
    )(segmax)


def _sc_gather(table, idx3, rows_per_w, row_w, out_rows, dtype):
    nchunks = rows_per_w // 128
    mesh = plsc.VectorSubcoreMesh(core_axis_name="c", subcore_axis_name="s")

    @functools.partial(
        pl.kernel,
        mesh=mesh,
        out_type=jax.ShapeDtypeStruct((out_rows, row_w), dtype),
        scratch_types=[
            pltpu.VMEM((nchunks, 128), jnp.int32),
            pltpu.VMEM((128, row_w), dtype),
            pltpu.SemaphoreType.DMA,
        ],
    )
    def k(table_hbm, idx_hbm, out_hbm, idx_v, rows_v, sem):
        wid = lax.axis_index("s") * 2 + lax.axis_index("c")
        pltpu.sync_copy(idx_hbm.at[wid], idx_v)
        base = wid * rows_per_w
        for c in range(nchunks):
            pltpu.async_copy(table_hbm.at[idx_v.at[c]], rows_v, sem).wait()
            pltpu.sync_copy(rows_v, out_hbm.at[pl.ds(base + c * 128, 128)])

    return k(table, idx3)


def _s4_body(cand_ref, seg_ref, w_ref, idx_ref):
    v = cand_ref[...]
    segsel = seg_ref[...]
    seg3 = segsel[:, :, None]
    lane2 = lax.broadcasted_iota(jnp.int32, (QB4, SEG), 1)
    lane3 = lax.broadcasted_iota(jnp.int32, (QB4, K, SEG), 2)
    kiota = lax.broadcasted_iota(jnp.int32, (QB4, K), 1)
    m2 = jnp.max(v, axis=2)
    val_acc = jnp.zeros((QB4, K), jnp.float32)
    idx_acc = jnp.zeros((QB4, K), jnp.int32)
    for k in range(K):
        m = jnp.max(m2, axis=1, keepdims=True)
        jsel = jnp.min(jnp.where(m2 == m, segsel, IBIG), axis=1,
                       keepdims=True)
        j_oh = segsel == jsel
        j_oh3 = seg3 == jsel[:, :, None]
        vs = jnp.max(jnp.where(j_oh3, v, NEGF), axis=1)
        pos = jnp.min(jnp.where(vs == m, lane2, IBIG), axis=1,
                      keepdims=True)
        v = jnp.where(j_oh3 & (lane3 == pos[:, :, None]), NEGF, v)
        vs = jnp.where(lane2 == pos, NEGF, vs)
        m2 = jnp.where(j_oh, jnp.max(vs, axis=1, keepdims=True), m2)
        val_acc = jnp.where(kiota == k, m, val_acc)
        idx_acc = jnp.where(kiota == k, jsel * SEG + pos, idx_acc)
    e = jnp.exp(val_acc - jnp.max(val_acc, axis=1, keepdims=True))
    w_ref[...] = e / jnp.sum(e, axis=1, keepdims=True)
    idx_ref[...] = idx_acc


def _stage4(cand, segsel):
    return pl.pallas_call(
        _s4_body,
        grid=(Q // QB4,),
        in_specs=[
            pl.BlockSpec((QB4, K, SEG), lambda b: (b, 0, 0)),
            pl.BlockSpec((QB4, K), lambda b: (b, 0)),
        ],
        out_specs=[
            pl.BlockSpec((QB4, K), lambda b: (b, 0)),
            pl.BlockSpec((QB4, K), lambda b: (b, 0)),
        ],
        out_shape=[
            jax.ShapeDtypeStruct((Q, K), jnp.float32),
            jax.ShapeDtypeStruct((Q, K), jnp.int32),
        ],
    )(cand, segsel)


def _s6_body(w_ref, g_ref, wout_ref, out_ref):
    w = w_ref[...]
    g = g_ref[...]
    agg = jnp.sum(w[:, :, None] * g, axis=1)
    out_ref[...] = lax.dot_general(agg, wout_ref[...],
                                   (((1,), (1,)), ((), ())),
                                   preferred_element_type=jnp.float32)


def _stage6(weights, gathered, w_out):
    return pl.pallas_call(
        _s6_body,
        grid=(Q // QB6,),
        in_specs=[
            pl.BlockSpec((QB6, K), lambda b: (b, 0)),
            pl.BlockSpec((QB6, K, PD), lambda b: (b, 0, 0)),
            pl.BlockSpec((PD, PD), lambda b: (0, 0)),
        ],
        out_specs=pl.BlockSpec((QB6, PD), lambda b: (b, 0)),
        out_shape=jax.ShapeDtypeStruct((Q, PD), jnp.float32),
    )(weights, gathered, w_out)


def kernel(query, keys, pool, W_out):
    B, S, _ = query.shape
    q2 = query.reshape(Q, D)

    scores, segmax3 = _stage1(q2, keys)
    segmax = jnp.transpose(segmax3, (1, 0, 2)).reshape(Q, NSEG)

    segsel, absrow = _stage2(segmax)

    table = scores.reshape(Q * NSEG, SEG)
    idx3 = absrow.reshape(NW, (Q * K // NW) // 128, 128)
    cand = _sc_gather(table, idx3, Q * K // NW, SEG, Q * K,
                      jnp.float32)

    weights, topidx = _stage4(cand.reshape(Q, K, SEG), segsel)

    pidx3 = topidx.reshape(NW, (Q * K // NW) // 128, 128)
    gathered = _sc_gather(pool, pidx3, Q * K // NW, PD, Q * K, jnp.float32)

    out = _stage6(weights, gathered.reshape(Q, K, PD), W_out)
    return out.reshape(B, S, PD)
